# baseline (speedup 1.0000x reference)
import jax
import jax.numpy as jnp
from jax import lax
from jax.experimental import pallas as pl
from jax.experimental.pallas import tpu as pltpu
from jax.experimental.pallas import tpu_sc as plsc

B = 8
H = 512
W = 512
HW = H * W
BHW = B * HW
NC = 2
NS = 16
P = HW // NS
BPC = B // NC
ZB = 8192
RB = 128
NR = H // RB
G = RB * W // 128


def _prep_body(flow_ref, invd_ref, idx_ref, wx_ref, wy_ref, w_ref):
    r = pl.program_id(1)
    fx = flow_ref[0, 0].reshape(G, 128)
    fy = flow_ref[0, 1].reshape(G, 128)
    dv = invd_ref[0, 0].reshape(G, 128)
    gi = lax.broadcasted_iota(jnp.int32, (G, 128), 0)
    li = lax.broadcasted_iota(jnp.int32, (G, 128), 1)
    xi = ((gi & 3) << 7) + li
    yi = (gi >> 2) + r * RB
    tx = jnp.round(xi.astype(jnp.float32) - fx)
    ty = jnp.round(yi.astype(jnp.float32) - fy)
    inr = (tx >= 0.0) & (tx < float(W)) & (ty >= 0.0) & (ty < float(H))
    tgt = tx.astype(jnp.int32) + ty.astype(jnp.int32) * W
    own = xi + yi * W
    w = jnp.where(inr, dv, 0.0)
    idx_ref[...] = jnp.where(inr, tgt, own)
    wx_ref[...] = fx * w
    wy_ref[...] = fy * w
    w_ref[...] = w


_prep = pl.pallas_call(
    _prep_body,
    grid=(B, NR),
    in_specs=[
        pl.BlockSpec((1, 2, RB, W), lambda b, r: (b, 0, r, 0)),
        pl.BlockSpec((1, 1, RB, W), lambda b, r: (b, 0, r, 0)),
    ],
    out_specs=[pl.BlockSpec((G, 128), lambda b, r: (b * NR + r, 0))] * 4,
    out_shape=[
        jax.ShapeDtypeStruct((BHW // 128, 128), jnp.int32),
        jax.ShapeDtypeStruct((BHW // 128, 128), jnp.float32),
        jax.ShapeDtypeStruct((BHW // 128, 128), jnp.float32),
        jax.ShapeDtypeStruct((BHW // 128, 128), jnp.float32),
    ],
)


def _sc_body(idx_hbm, wx_hbm, wy_hbm, w_hbm, ax_hbm, ay_hbm, aw_hbm,
             idx_v, vx_v, vy_v, vw_v, zb_v, acc_x, acc_y, acc_w):
    c = lax.axis_index("c")
    s = lax.axis_index("s")
    base = s * P

    def _zb(i, carry):
        zb_v[pl.ds(pl.multiple_of(i * 16, 16), 16)] = jnp.zeros((16,), jnp.float32)
        return carry

    lax.fori_loop(0, ZB // 16, _zb, 0, unroll=4)

    for k in range(BPC):
        b = c * BPC + k
        goff = b * HW + base
        for acc in (acc_x, acc_y, acc_w):
            pltpu.sync_copy(zb_v, acc.at[pl.ds(base, ZB)])
            pltpu.sync_copy(zb_v, acc.at[pl.ds(base + ZB, ZB)])
        pltpu.sync_copy(idx_hbm.at[pl.ds(goff, P)], idx_v)
        pltpu.sync_copy(wx_hbm.at[pl.ds(goff, P)], vx_v)
        pltpu.sync_copy(wy_hbm.at[pl.ds(goff, P)], vy_v)
        pltpu.sync_copy(w_hbm.at[pl.ds(goff, P)], vw_v)
        plsc.subcore_barrier()

        pltpu.sync_copy(vx_v, acc_x.at[idx_v], add=True)
        pltpu.sync_copy(vy_v, acc_y.at[idx_v], add=True)
        pltpu.sync_copy(vw_v, acc_w.at[idx_v], add=True)
        plsc.subcore_barrier()

        pltpu.sync_copy(acc_x.at[pl.ds(base, P)], ax_hbm.at[pl.ds(goff, P)])
        pltpu.sync_copy(acc_y.at[pl.ds(base, P)], ay_hbm.at[pl.ds(goff, P)])
        pltpu.sync_copy(acc_w.at[pl.ds(base, P)], aw_hbm.at[pl.ds(goff, P)])


def _build_sc_kernel():
    return pl.kernel(
        _sc_body,
        out_type=(
            jax.ShapeDtypeStruct((BHW,), jnp.float32),
            jax.ShapeDtypeStruct((BHW,), jnp.float32),
            jax.ShapeDtypeStruct((BHW,), jnp.float32),
        ),
        mesh=plsc.VectorSubcoreMesh(
            core_axis_name="c", subcore_axis_name="s", num_cores=NC, num_subcores=NS
        ),
        scratch_types=[
            pltpu.VMEM((P,), jnp.int32),
            pltpu.VMEM((P,), jnp.float32),
            pltpu.VMEM((P,), jnp.float32),
            pltpu.VMEM((P,), jnp.float32),
            pltpu.VMEM((ZB,), jnp.float32),
            pltpu.VMEM_SHARED((HW,), jnp.float32),
            pltpu.VMEM_SHARED((HW,), jnp.float32),
            pltpu.VMEM_SHARED((HW,), jnp.float32),
        ],
    )


def _fin_body(ax_ref, ay_ref, aw_ref, out_ref):
    ax = ax_ref[...]
    ay = ay_ref[...]
    aw = aw_ref[...]
    inv = jnp.where(ax != 0.0, 1.0 / (aw + 1e-7), 0.0)
    out_ref[0, 0] = (ax * inv).reshape(H, W)
    out_ref[0, 1] = (ay * inv).reshape(H, W)


_finalize = pl.pallas_call(
    _fin_body,
    grid=(B,),
    in_specs=[pl.BlockSpec((HW // 128, 128), lambda b: (b, 0))] * 3,
    out_specs=pl.BlockSpec((1, 2, H, W), lambda b: (b, 0, 0, 0)),
    out_shape=jax.ShapeDtypeStruct((B, 2, H, W), jnp.float32),
)


def kernel(flow, inv_depth):
    idx, wx, wy, w = _prep(flow, inv_depth)
    ax, ay, aw = _build_sc_kernel()(
        idx.reshape(BHW),
        wx.reshape(BHW),
        wy.reshape(BHW),
        w.reshape(BHW),
    )
    return _finalize(
        ax.reshape(BHW // 128, 128),
        ay.reshape(BHW // 128, 128),
        aw.reshape(BHW // 128, 128),
    )


# speedup vs baseline: 1.0870x; 1.0870x over previous
import jax
import jax.numpy as jnp
from jax import lax
from jax.experimental import pallas as pl
from jax.experimental.pallas import tpu as pltpu
from jax.experimental.pallas import tpu_sc as plsc

B = 8
H = 512
W = 512
HW = H * W
BHW = B * HW
NC = 2
NS = 16
P = HW // NS
BPC = B // NC
ZB = 8192
RB = 512
NR = H // RB
G = RB * W // 128


def _prep_body(flow_ref, invd_ref, idx_ref, wx_ref, wy_ref, w_ref):
    r = pl.program_id(1)
    fx = flow_ref[0, 0].reshape(G, 128)
    fy = flow_ref[0, 1].reshape(G, 128)
    dv = invd_ref[0, 0].reshape(G, 128)
    gi = lax.broadcasted_iota(jnp.int32, (G, 128), 0)
    li = lax.broadcasted_iota(jnp.int32, (G, 128), 1)
    xi = ((gi & 3) << 7) + li
    yi = (gi >> 2) + r * RB
    tx = jnp.round(xi.astype(jnp.float32) - fx)
    ty = jnp.round(yi.astype(jnp.float32) - fy)
    inr = (tx >= 0.0) & (tx < float(W)) & (ty >= 0.0) & (ty < float(H))
    tgt = tx.astype(jnp.int32) + ty.astype(jnp.int32) * W
    own = xi + yi * W
    w = jnp.where(inr, dv, 0.0)
    idx_ref[...] = jnp.where(inr, tgt, own)
    wx_ref[...] = fx * w
    wy_ref[...] = fy * w
    w_ref[...] = w


_prep = pl.pallas_call(
    _prep_body,
    grid=(B, NR),
    in_specs=[
        pl.BlockSpec((1, 2, RB, W), lambda b, r: (b, 0, r, 0)),
        pl.BlockSpec((1, 1, RB, W), lambda b, r: (b, 0, r, 0)),
    ],
    out_specs=[pl.BlockSpec((G, 128), lambda b, r: (b * NR + r, 0))] * 4,
    out_shape=[
        jax.ShapeDtypeStruct((BHW // 128, 128), jnp.int32),
        jax.ShapeDtypeStruct((BHW // 128, 128), jnp.float32),
        jax.ShapeDtypeStruct((BHW // 128, 128), jnp.float32),
        jax.ShapeDtypeStruct((BHW // 128, 128), jnp.float32),
    ],
)


def _sc_body(idx_hbm, wx_hbm, wy_hbm, w_hbm, ax_hbm, ay_hbm, aw_hbm,
             idx_v, vx_v, vy_v, vw_v, zb_v, acc_x, acc_y, acc_w):
    c = lax.axis_index("c")
    s = lax.axis_index("s")
    base = s * P

    def _zb(i, carry):
        zb_v[pl.ds(pl.multiple_of(i * 16, 16), 16)] = jnp.zeros((16,), jnp.float32)
        return carry

    lax.fori_loop(0, ZB // 16, _zb, 0, unroll=4)

    for k in range(BPC):
        b = c * BPC + k
        goff = b * HW + base
        for acc in (acc_x, acc_y, acc_w):
            pltpu.sync_copy(zb_v, acc.at[pl.ds(base, ZB)])
            pltpu.sync_copy(zb_v, acc.at[pl.ds(base + ZB, ZB)])
        pltpu.sync_copy(idx_hbm.at[pl.ds(goff, P)], idx_v)
        pltpu.sync_copy(wx_hbm.at[pl.ds(goff, P)], vx_v)
        pltpu.sync_copy(wy_hbm.at[pl.ds(goff, P)], vy_v)
        pltpu.sync_copy(w_hbm.at[pl.ds(goff, P)], vw_v)
        plsc.subcore_barrier()

        pltpu.sync_copy(vx_v, acc_x.at[idx_v], add=True)
        pltpu.sync_copy(vy_v, acc_y.at[idx_v], add=True)
        pltpu.sync_copy(vw_v, acc_w.at[idx_v], add=True)
        plsc.subcore_barrier()

        pltpu.sync_copy(acc_x.at[pl.ds(base, P)], ax_hbm.at[pl.ds(goff, P)])
        pltpu.sync_copy(acc_y.at[pl.ds(base, P)], ay_hbm.at[pl.ds(goff, P)])
        pltpu.sync_copy(acc_w.at[pl.ds(base, P)], aw_hbm.at[pl.ds(goff, P)])


def _build_sc_kernel():
    return pl.kernel(
        _sc_body,
        out_type=(
            jax.ShapeDtypeStruct((BHW,), jnp.float32),
            jax.ShapeDtypeStruct((BHW,), jnp.float32),
            jax.ShapeDtypeStruct((BHW,), jnp.float32),
        ),
        mesh=plsc.VectorSubcoreMesh(
            core_axis_name="c", subcore_axis_name="s", num_cores=NC, num_subcores=NS
        ),
        scratch_types=[
            pltpu.VMEM((P,), jnp.int32),
            pltpu.VMEM((P,), jnp.float32),
            pltpu.VMEM((P,), jnp.float32),
            pltpu.VMEM((P,), jnp.float32),
            pltpu.VMEM((ZB,), jnp.float32),
            pltpu.VMEM_SHARED((HW,), jnp.float32),
            pltpu.VMEM_SHARED((HW,), jnp.float32),
            pltpu.VMEM_SHARED((HW,), jnp.float32),
        ],
    )


def _fin_body(ax_ref, ay_ref, aw_ref, out_ref):
    ax = ax_ref[...]
    ay = ay_ref[...]
    aw = aw_ref[...]
    inv = jnp.where(ax != 0.0, 1.0 / (aw + 1e-7), 0.0)
    out_ref[0, 0] = (ax * inv).reshape(H, W)
    out_ref[0, 1] = (ay * inv).reshape(H, W)


_finalize = pl.pallas_call(
    _fin_body,
    grid=(B,),
    in_specs=[pl.BlockSpec((HW // 128, 128), lambda b: (b, 0))] * 3,
    out_specs=pl.BlockSpec((1, 2, H, W), lambda b: (b, 0, 0, 0)),
    out_shape=jax.ShapeDtypeStruct((B, 2, H, W), jnp.float32),
)


def kernel(flow, inv_depth):
    idx, wx, wy, w = _prep(flow, inv_depth)
    ax, ay, aw = _build_sc_kernel()(
        idx.reshape(BHW),
        wx.reshape(BHW),
        wy.reshape(BHW),
        w.reshape(BHW),
    )
    return _finalize(
        ax.reshape(BHW // 128, 128),
        ay.reshape(BHW // 128, 128),
        aw.reshape(BHW // 128, 128),
    )


# speedup vs baseline: 1.1349x; 1.0440x over previous
import jax
import jax.numpy as jnp
from jax import lax
from jax.experimental import pallas as pl
from jax.experimental.pallas import tpu as pltpu
from jax.experimental.pallas import tpu_sc as plsc

B = 8
H = 512
W = 512
HW = H * W
NC = 2
NS = 16
P = HW // NS
ZB = 8192
RB = 512
G = RB * W // 128
BH = B // 2
HHW = BH * HW


def _prep_body(flow_ref, invd_ref, idx_ref, wx_ref, wy_ref, w_ref):
    fx = flow_ref[0, 0].reshape(G, 128)
    fy = flow_ref[0, 1].reshape(G, 128)
    dv = invd_ref[0, 0].reshape(G, 128)
    gi = lax.broadcasted_iota(jnp.int32, (G, 128), 0)
    li = lax.broadcasted_iota(jnp.int32, (G, 128), 1)
    xi = ((gi & 3) << 7) + li
    yi = gi >> 2
    tx = jnp.round(xi.astype(jnp.float32) - fx)
    ty = jnp.round(yi.astype(jnp.float32) - fy)
    inr = (tx >= 0.0) & (tx < float(W)) & (ty >= 0.0) & (ty < float(H))
    tgt = tx.astype(jnp.int32) + ty.astype(jnp.int32) * W
    own = xi + yi * W
    w = jnp.where(inr, dv, 0.0)
    idx_ref[...] = jnp.where(inr, tgt, own)
    wx_ref[...] = fx * w
    wy_ref[...] = fy * w
    w_ref[...] = w


def _make_prep(h):
    return pl.pallas_call(
        _prep_body,
        grid=(BH,),
        in_specs=[
            pl.BlockSpec((1, 2, RB, W), lambda i: (h * BH + i, 0, 0, 0)),
            pl.BlockSpec((1, 1, RB, W), lambda i: (h * BH + i, 0, 0, 0)),
        ],
        out_specs=[pl.BlockSpec((G, 128), lambda i: (i, 0))] * 4,
        out_shape=[
            jax.ShapeDtypeStruct((HHW // 128, 128), jnp.int32),
            jax.ShapeDtypeStruct((HHW // 128, 128), jnp.float32),
            jax.ShapeDtypeStruct((HHW // 128, 128), jnp.float32),
            jax.ShapeDtypeStruct((HHW // 128, 128), jnp.float32),
        ],
    )


def _sc_body(idx_hbm, wx_hbm, wy_hbm, w_hbm, ax_hbm, ay_hbm, aw_hbm,
             idx_v, vx_v, vy_v, vw_v, zb_v, acc_x, acc_y, acc_w):
    c = lax.axis_index("c")
    s = lax.axis_index("s")
    base = s * P

    def _zb(i, carry):
        zb_v[pl.ds(pl.multiple_of(i * 16, 16), 16)] = jnp.zeros((16,), jnp.float32)
        return carry

    lax.fori_loop(0, ZB // 16, _zb, 0, unroll=4)

    for k in range(BH // NC):
        pos = c * (BH // NC) + k
        goff = pos * HW + base
        for acc in (acc_x, acc_y, acc_w):
            pltpu.sync_copy(zb_v, acc.at[pl.ds(base, ZB)])
            pltpu.sync_copy(zb_v, acc.at[pl.ds(base + ZB, ZB)])
        pltpu.sync_copy(idx_hbm.at[pl.ds(goff, P)], idx_v)
        pltpu.sync_copy(wx_hbm.at[pl.ds(goff, P)], vx_v)
        pltpu.sync_copy(wy_hbm.at[pl.ds(goff, P)], vy_v)
        pltpu.sync_copy(w_hbm.at[pl.ds(goff, P)], vw_v)
        plsc.subcore_barrier()

        pltpu.sync_copy(vx_v, acc_x.at[idx_v], add=True)
        pltpu.sync_copy(vy_v, acc_y.at[idx_v], add=True)
        pltpu.sync_copy(vw_v, acc_w.at[idx_v], add=True)
        plsc.subcore_barrier()

        pltpu.sync_copy(acc_x.at[pl.ds(base, P)], ax_hbm.at[pl.ds(goff, P)])
        pltpu.sync_copy(acc_y.at[pl.ds(base, P)], ay_hbm.at[pl.ds(goff, P)])
        pltpu.sync_copy(acc_w.at[pl.ds(base, P)], aw_hbm.at[pl.ds(goff, P)])


def _build_sc_kernel():
    return pl.kernel(
        _sc_body,
        out_type=(
            jax.ShapeDtypeStruct((HHW,), jnp.float32),
            jax.ShapeDtypeStruct((HHW,), jnp.float32),
            jax.ShapeDtypeStruct((HHW,), jnp.float32),
        ),
        mesh=plsc.VectorSubcoreMesh(
            core_axis_name="c", subcore_axis_name="s", num_cores=NC, num_subcores=NS
        ),
        scratch_types=[
            pltpu.VMEM((P,), jnp.int32),
            pltpu.VMEM((P,), jnp.float32),
            pltpu.VMEM((P,), jnp.float32),
            pltpu.VMEM((P,), jnp.float32),
            pltpu.VMEM((ZB,), jnp.float32),
            pltpu.VMEM_SHARED((HW,), jnp.float32),
            pltpu.VMEM_SHARED((HW,), jnp.float32),
            pltpu.VMEM_SHARED((HW,), jnp.float32),
        ],
    )


def _fin_compute(ax_ref, ay_ref, aw_ref, out_ref):
    ax = ax_ref[...]
    ay = ay_ref[...]
    aw = aw_ref[...]
    inv = jnp.where(ax != 0.0, 1.0 / (aw + 1e-7), 0.0)
    out_ref[0, 0] = (ax * inv).reshape(H, W)
    out_ref[0, 1] = (ay * inv).reshape(H, W)


def _fin_body_first(ax_ref, ay_ref, aw_ref, out_ref):
    _fin_compute(ax_ref, ay_ref, aw_ref, out_ref)


def _fin_body_second(prev_ref, ax_ref, ay_ref, aw_ref, out_ref):
    del prev_ref
    _fin_compute(ax_ref, ay_ref, aw_ref, out_ref)


_ACC_SPEC = pl.BlockSpec((HW // 128, 128), lambda i: (i, 0))

_fin_first = pl.pallas_call(
    _fin_body_first,
    grid=(BH,),
    in_specs=[_ACC_SPEC] * 3,
    out_specs=pl.BlockSpec((1, 2, H, W), lambda i: (i, 0, 0, 0)),
    out_shape=jax.ShapeDtypeStruct((B, 2, H, W), jnp.float32),
)

_fin_second = pl.pallas_call(
    _fin_body_second,
    grid=(BH,),
    in_specs=[pl.BlockSpec(memory_space=pltpu.MemorySpace.HBM)] + [_ACC_SPEC] * 3,
    out_specs=pl.BlockSpec((1, 2, H, W), lambda i: (BH + i, 0, 0, 0)),
    out_shape=jax.ShapeDtypeStruct((B, 2, H, W), jnp.float32),
    input_output_aliases={0: 0},
)


def kernel(flow, inv_depth):
    sc = _build_sc_kernel()
    idx0, wx0, wy0, w0 = _make_prep(0)(flow, inv_depth)
    ax0, ay0, aw0 = sc(
        idx0.reshape(HHW), wx0.reshape(HHW), wy0.reshape(HHW), w0.reshape(HHW)
    )
    idx1, wx1, wy1, w1 = _make_prep(1)(flow, inv_depth)
    ax1, ay1, aw1 = sc(
        idx1.reshape(HHW), wx1.reshape(HHW), wy1.reshape(HHW), w1.reshape(HHW)
    )
    out = _fin_first(
        ax0.reshape(HHW // 128, 128),
        ay0.reshape(HHW // 128, 128),
        aw0.reshape(HHW // 128, 128),
    )
    return _fin_second(
        out,
        ax1.reshape(HHW // 128, 128),
        ay1.reshape(HHW // 128, 128),
        aw1.reshape(HHW // 128, 128),
    )


# speedup vs baseline: 1.1364x; 1.0014x over previous
import jax
import jax.numpy as jnp
from jax import lax
from jax.experimental import pallas as pl
from jax.experimental.pallas import tpu as pltpu
from jax.experimental.pallas import tpu_sc as plsc

B = 8
H = 512
W = 512
HW = H * W
NC = 2
NS = 16
P = HW // NS
ZB = 8192
RB = 512
G = RB * W // 128
BH = B // 2
HHW = BH * HW


def _prep_body(flow_ref, invd_ref, idx_ref, wx_ref, wy_ref, w_ref):
    fx = flow_ref[0, 0].reshape(G, 128)
    fy = flow_ref[0, 1].reshape(G, 128)
    dv = invd_ref[0, 0].reshape(G, 128)
    gi = lax.broadcasted_iota(jnp.int32, (G, 128), 0)
    li = lax.broadcasted_iota(jnp.int32, (G, 128), 1)
    xi = ((gi & 3) << 7) + li
    yi = gi >> 2
    tx = jnp.round(xi.astype(jnp.float32) - fx)
    ty = jnp.round(yi.astype(jnp.float32) - fy)
    inr = (tx >= 0.0) & (tx < float(W)) & (ty >= 0.0) & (ty < float(H))
    tgt = tx.astype(jnp.int32) + ty.astype(jnp.int32) * W
    own = xi + yi * W
    w = jnp.where(inr, dv, 0.0)
    idx_ref[...] = jnp.where(inr, tgt, own)
    wx_ref[...] = fx * w
    wy_ref[...] = fy * w
    w_ref[...] = w


def _make_prep(h):
    return pl.pallas_call(
        _prep_body,
        grid=(BH,),
        in_specs=[
            pl.BlockSpec((1, 2, RB, W), lambda i: (h * BH + i, 0, 0, 0)),
            pl.BlockSpec((1, 1, RB, W), lambda i: (h * BH + i, 0, 0, 0)),
        ],
        out_specs=[pl.BlockSpec((G, 128), lambda i: (i, 0))] * 4,
        out_shape=[
            jax.ShapeDtypeStruct((HHW // 128, 128), jnp.int32),
            jax.ShapeDtypeStruct((HHW // 128, 128), jnp.float32),
            jax.ShapeDtypeStruct((HHW // 128, 128), jnp.float32),
            jax.ShapeDtypeStruct((HHW // 128, 128), jnp.float32),
        ],
    )


def _sc_body(idx_hbm, wx_hbm, wy_hbm, w_hbm, ax_hbm, ay_hbm, aw_hbm,
             idx_v, vx_v, vy_v, vw_v, zb_v, acc_x, acc_y, acc_w, sem):
    c = lax.axis_index("c")
    s = lax.axis_index("s")
    base = s * P

    def _zb(i, carry):
        zb_v[pl.ds(pl.multiple_of(i * 16, 16), 16)] = jnp.zeros((16,), jnp.float32)
        return carry

    lax.fori_loop(0, ZB // 16, _zb, 0, unroll=4)

    for k in range(BH // NC):
        pos = c * (BH // NC) + k
        goff = pos * HW + base
        for acc in (acc_x, acc_y, acc_w):
            pltpu.sync_copy(zb_v, acc.at[pl.ds(base, ZB)])
            pltpu.sync_copy(zb_v, acc.at[pl.ds(base + ZB, ZB)])
        pltpu.sync_copy(idx_hbm.at[pl.ds(goff, P)], idx_v)
        pltpu.sync_copy(wx_hbm.at[pl.ds(goff, P)], vx_v)
        pltpu.sync_copy(wy_hbm.at[pl.ds(goff, P)], vy_v)
        pltpu.sync_copy(w_hbm.at[pl.ds(goff, P)], vw_v)
        plsc.subcore_barrier()

        hx = pltpu.async_copy(vx_v, acc_x.at[idx_v], sem, add=True)
        hy = pltpu.async_copy(vy_v, acc_y.at[idx_v], sem, add=True)
        hw_ = pltpu.async_copy(vw_v, acc_w.at[idx_v], sem, add=True)
        hx.wait()
        hy.wait()
        hw_.wait()
        plsc.subcore_barrier()

        pltpu.sync_copy(acc_x.at[pl.ds(base, P)], ax_hbm.at[pl.ds(goff, P)])
        pltpu.sync_copy(acc_y.at[pl.ds(base, P)], ay_hbm.at[pl.ds(goff, P)])
        pltpu.sync_copy(acc_w.at[pl.ds(base, P)], aw_hbm.at[pl.ds(goff, P)])


def _build_sc_kernel():
    return pl.kernel(
        _sc_body,
        out_type=(
            jax.ShapeDtypeStruct((HHW,), jnp.float32),
            jax.ShapeDtypeStruct((HHW,), jnp.float32),
            jax.ShapeDtypeStruct((HHW,), jnp.float32),
        ),
        mesh=plsc.VectorSubcoreMesh(
            core_axis_name="c", subcore_axis_name="s", num_cores=NC, num_subcores=NS
        ),
        scratch_types=[
            pltpu.VMEM((P,), jnp.int32),
            pltpu.VMEM((P,), jnp.float32),
            pltpu.VMEM((P,), jnp.float32),
            pltpu.VMEM((P,), jnp.float32),
            pltpu.VMEM((ZB,), jnp.float32),
            pltpu.VMEM_SHARED((HW,), jnp.float32),
            pltpu.VMEM_SHARED((HW,), jnp.float32),
            pltpu.VMEM_SHARED((HW,), jnp.float32),
            pltpu.SemaphoreType.DMA,
        ],
    )


def _fin_compute(ax_ref, ay_ref, aw_ref, out_ref):
    ax = ax_ref[...]
    ay = ay_ref[...]
    aw = aw_ref[...]
    inv = jnp.where(ax != 0.0, 1.0 / (aw + 1e-7), 0.0)
    out_ref[0, 0] = (ax * inv).reshape(H, W)
    out_ref[0, 1] = (ay * inv).reshape(H, W)


def _fin_body_first(ax_ref, ay_ref, aw_ref, out_ref):
    _fin_compute(ax_ref, ay_ref, aw_ref, out_ref)


def _fin_body_second(prev_ref, ax_ref, ay_ref, aw_ref, out_ref):
    del prev_ref
    _fin_compute(ax_ref, ay_ref, aw_ref, out_ref)


_ACC_SPEC = pl.BlockSpec((HW // 128, 128), lambda i: (i, 0))

_fin_first = pl.pallas_call(
    _fin_body_first,
    grid=(BH,),
    in_specs=[_ACC_SPEC] * 3,
    out_specs=pl.BlockSpec((1, 2, H, W), lambda i: (i, 0, 0, 0)),
    out_shape=jax.ShapeDtypeStruct((B, 2, H, W), jnp.float32),
)

_fin_second = pl.pallas_call(
    _fin_body_second,
    grid=(BH,),
    in_specs=[pl.BlockSpec(memory_space=pltpu.MemorySpace.HBM)] + [_ACC_SPEC] * 3,
    out_specs=pl.BlockSpec((1, 2, H, W), lambda i: (BH + i, 0, 0, 0)),
    out_shape=jax.ShapeDtypeStruct((B, 2, H, W), jnp.float32),
    input_output_aliases={0: 0},
)


def kernel(flow, inv_depth):
    sc = _build_sc_kernel()
    idx0, wx0, wy0, w0 = _make_prep(0)(flow, inv_depth)
    ax0, ay0, aw0 = sc(
        idx0.reshape(HHW), wx0.reshape(HHW), wy0.reshape(HHW), w0.reshape(HHW)
    )
    idx1, wx1, wy1, w1 = _make_prep(1)(flow, inv_depth)
    ax1, ay1, aw1 = sc(
        idx1.reshape(HHW), wx1.reshape(HHW), wy1.reshape(HHW), w1.reshape(HHW)
    )
    out = _fin_first(
        ax0.reshape(HHW // 128, 128),
        ay0.reshape(HHW // 128, 128),
        aw0.reshape(HHW // 128, 128),
    )
    return _fin_second(
        out,
        ax1.reshape(HHW // 128, 128),
        ay1.reshape(HHW // 128, 128),
        aw1.reshape(HHW // 128, 128),
    )
